# baseline (speedup 1.0000x reference)
import functools

import jax
import jax.numpy as jnp
from jax import lax
from jax.experimental import pallas as pl
from jax.experimental.pallas import tpu as pltpu
from jax.experimental.pallas import tpu_sc as plsc

_NBUF = 4
_CH = 16


def _sc_geometry():
    try:
        info = plsc.get_sparse_core_info()
        return info.num_cores, info.num_subcores
    except Exception:
        return 2, 16


@functools.lru_cache(maxsize=None)
def _build(vocab: int, d: int, n: int):
    nc, ns = _sc_geometry()
    nw = nc * ns
    n_per_w = n // nw
    ch = _CH
    nch = n_per_w // ch
    assert n_per_w % ch == 0 and nch % _NBUF == 0

    mesh = plsc.VectorSubcoreMesh(core_axis_name="c", subcore_axis_name="s")

    @functools.partial(
        pl.kernel,
        mesh=mesh,
        out_type=jax.ShapeDtypeStruct((n, d), jnp.float32),
        scratch_types=[
            pltpu.VMEM((nch, ch), jnp.int32),
            *([pltpu.VMEM((ch, d), jnp.float32)] * _NBUF),
            *([pltpu.SemaphoreType.DMA] * _NBUF),
            *([pltpu.SemaphoreType.DMA] * _NBUF),
        ],
    )
    def gather_kernel(table_hbm, idx_hbm, out_hbm, idx_v, *rest):
        bufs = rest[:_NBUF]
        gsems = rest[_NBUF : 2 * _NBUF]
        wsems = rest[2 * _NBUF :]
        wid = lax.axis_index("s") * nc + lax.axis_index("c")
        base = wid * n_per_w

        pltpu.sync_copy(idx_hbm.at[wid], idx_v)

        def gather(c, b):
            pltpu.async_copy(table_hbm.at[idx_v.at[c]], bufs[b], gsems[b])

        def wait_gather(c, b):
            pltpu.make_async_copy(
                table_hbm.at[idx_v.at[c]], bufs[b], gsems[b]
            ).wait()

        def write(c, b):
            pltpu.async_copy(
                bufs[b], out_hbm.at[pl.ds(base + c * ch, ch)], wsems[b]
            )

        def wait_write(c, b):
            pltpu.make_async_copy(
                bufs[b], out_hbm.at[pl.ds(base + c * ch, ch)], wsems[b]
            ).wait()

        for b in range(_NBUF - 1):
            gather(b, b)

        def step(g, _):
            for b in range(_NBUF):
                c = g * _NBUF + b
                wait_gather(c, b)
                write(c, b)
                nxt = c + _NBUF - 1
                bn = (b + _NBUF - 1) % _NBUF

                @pl.when(nxt < nch)
                def _():
                    @pl.when(c >= 1)
                    def _():
                        wait_write(c - 1, bn)

                    gather(nxt, bn)

            return ()

        lax.fori_loop(0, nch // _NBUF, step, (), unroll=False)

        for k in range(nch - _NBUF, nch):
            wait_write(k, k % _NBUF)

    return gather_kernel


def kernel(input_ids, embed_tokens):
    vocab, d = embed_tokens.shape
    nc, ns = _sc_geometry()
    nw = nc * ns
    ids = input_ids.reshape(-1).astype(jnp.int32)
    n = ids.shape[0]
    idx = ids.reshape(nw, (n // nw) // _CH, _CH)
    out = _build(vocab, d, n)(embed_tokens, idx)
    return out.reshape(*input_ids.shape, d)


# speedup vs baseline: 1.0022x; 1.0022x over previous
import functools

import jax
import jax.numpy as jnp
from jax import lax
from jax.experimental import pallas as pl
from jax.experimental.pallas import tpu as pltpu
from jax.experimental.pallas import tpu_sc as plsc

_NBUF = 4
_CH = 16


def _sc_geometry():
    try:
        info = plsc.get_sparse_core_info()
        return info.num_cores, info.num_subcores
    except Exception:
        return 2, 16


@functools.lru_cache(maxsize=None)
def _build(vocab: int, d: int, n: int):
    nc, ns = _sc_geometry()
    nw = nc * ns
    n_per_w = n // nw
    ch = _CH
    nch = n_per_w // ch
    assert n_per_w % ch == 0 and nch % _NBUF == 0

    mesh = plsc.VectorSubcoreMesh(core_axis_name="c", subcore_axis_name="s")

    @functools.partial(
        pl.kernel,
        mesh=mesh,
        out_type=jax.ShapeDtypeStruct((n, d), jnp.float32),
        scratch_types=[
            pltpu.VMEM((n_per_w,), jnp.int32),
            *([pltpu.VMEM((ch, d), jnp.float32)] * _NBUF),
            *([pltpu.SemaphoreType.DMA] * _NBUF),
            *([pltpu.SemaphoreType.DMA] * _NBUF),
        ],
    )
    def gather_kernel(table_hbm, idx_hbm, out_hbm, idx_v, *rest):
        bufs = rest[:_NBUF]
        gsems = rest[_NBUF : 2 * _NBUF]
        wsems = rest[2 * _NBUF :]
        wid = lax.axis_index("s") * nc + lax.axis_index("c")
        base = wid * n_per_w
        seq = idx_hbm.shape[1]
        w_per_row = seq // n_per_w

        pltpu.sync_copy(
            idx_hbm.at[wid // w_per_row, pl.ds((wid % w_per_row) * n_per_w, n_per_w)],
            idx_v,
        )

        def gather(c, b):
            pltpu.async_copy(
                table_hbm.at[idx_v.at[pl.ds(c * ch, ch)]], bufs[b], gsems[b]
            )

        def wait_gather(c, b):
            pltpu.make_async_copy(
                table_hbm.at[idx_v.at[pl.ds(c * ch, ch)]], bufs[b], gsems[b]
            ).wait()

        def write(c, b):
            pltpu.async_copy(
                bufs[b], out_hbm.at[pl.ds(base + c * ch, ch)], wsems[b]
            )

        def wait_write(c, b):
            pltpu.make_async_copy(
                bufs[b], out_hbm.at[pl.ds(base + c * ch, ch)], wsems[b]
            ).wait()

        for b in range(_NBUF - 1):
            gather(b, b)

        def step(g, _):
            for b in range(_NBUF):
                c = g * _NBUF + b
                wait_gather(c, b)
                write(c, b)
                nxt = c + _NBUF - 1
                bn = (b + _NBUF - 1) % _NBUF

                @pl.when(nxt < nch)
                def _():
                    @pl.when(c >= 1)
                    def _():
                        wait_write(c - 1, bn)

                    gather(nxt, bn)

            return ()

        lax.fori_loop(0, nch // _NBUF, step, (), unroll=False)

        for k in range(nch - _NBUF, nch):
            wait_write(k, k % _NBUF)

    return gather_kernel


def kernel(input_ids, embed_tokens):
    vocab, d = embed_tokens.shape
    n = input_ids.shape[0] * input_ids.shape[1]
    out = _build(vocab, d, n)(embed_tokens, input_ids.astype(jnp.int32))
    return out.reshape(*input_ids.shape, d)
